# baseline (speedup 1.0000x reference)
import functools

import jax
import jax.numpy as jnp
from jax import lax
from jax.experimental import pallas as pl
from jax.experimental.pallas import tpu as pltpu
from jax.experimental.pallas import tpu_sc as plsc

N = 10000
D = 128
HC = D // 2
E = 320000
NUM_CORES = 2
NUM_SUBCORES = 16
CHUNK = 80
NCHUNK = 256
EDGES_PER_T = NCHUNK * CHUNK
EPAD = NUM_SUBCORES * EDGES_PER_T
NPAD = 10112
ROWS_PER_TILE = NPAD // NUM_SUBCORES
LANES = 16
STAGE_CHUNKS = 16
NSTAGE = NCHUNK // STAGE_CHUNKS
XROWS_PER_TILE = 624
ZROWS = CHUNK


def _sc_scatter_halves(src, dst, rf, xcols):
    mesh = plsc.VectorSubcoreMesh(
        core_axis_name="c", subcore_axis_name="s",
        num_cores=NUM_CORES, num_subcores=NUM_SUBCORES)

    @functools.partial(
        pl.kernel,
        out_type=jax.ShapeDtypeStruct((NUM_CORES, NPAD, HC), jnp.float32),
        mesh=mesh,
        scratch_types=[
            pltpu.VMEM_SHARED((NPAD, HC), jnp.float32),
            pltpu.VMEM_SHARED((N, HC), jnp.float32),
            pltpu.VMEM((STAGE_CHUNKS, CHUNK), jnp.int32),
            pltpu.VMEM((STAGE_CHUNKS, CHUNK), jnp.int32),
            pltpu.VMEM((STAGE_CHUNKS, CHUNK), jnp.float32),
            pltpu.VMEM((CHUNK, HC), jnp.float32),
            pltpu.VMEM((CHUNK, HC), jnp.float32),
            pltpu.VMEM((CHUNK, HC), jnp.float32),
            pltpu.VMEM((CHUNK, HC), jnp.float32),
            pltpu.SemaphoreType.DMA,
            pltpu.SemaphoreType.DMA,
            pltpu.SemaphoreType.DMA,
            pltpu.SemaphoreType.DMA,
        ],
    )
    def k(src_hbm, dst_hbm, rf_hbm, xc_hbm, out_hbm,
          acc_sh, x_sh, srcv, dstv, rfv, rows0, rows1, rows2, rows3,
          semS, sem0, sem1, semW):
        cid = lax.axis_index("c")
        sid = lax.axis_index("s")

        zeros16 = jnp.zeros((LANES,), jnp.float32)

        def zrow(i, carry):
            for j in range(HC // LANES):
                rows0[i, pl.ds(j * LANES, LANES)] = zeros16
            return carry

        lax.fori_loop(0, ZROWS, zrow, 0)
        base_row = sid * ROWS_PER_TILE
        for t in range(ROWS_PER_TILE // ZROWS):
            pltpu.sync_copy(rows0, acc_sh.at[pl.ds(base_row + t * ZROWS, ZROWS)])
        pltpu.sync_copy(
            rows0.at[pl.ds(0, ROWS_PER_TILE % ZROWS)],
            acc_sh.at[pl.ds(base_row + (ROWS_PER_TILE // ZROWS) * ZROWS,
                            ROWS_PER_TILE % ZROWS)])

        xsl = pl.ds(sid * XROWS_PER_TILE, XROWS_PER_TILE)
        pltpu.sync_copy(xc_hbm.at[cid, xsl], x_sh.at[xsl])

        @pl.when(sid == 0)
        def _():
            tail = pl.ds(NUM_SUBCORES * XROWS_PER_TILE,
                         N - NUM_SUBCORES * XROWS_PER_TILE)
            pltpu.sync_copy(xc_hbm.at[cid, tail], x_sh.at[tail])

        plsc.subcore_barrier()

        def scale(rows, c):
            for g in range(CHUNK // LANES):
                wv = rfv[c, pl.ds(g * LANES, LANES)]
                for l in range(LANES):
                    w = wv[l]
                    i = g * LANES + l
                    for j in range(HC // LANES):
                        sl = pl.ds(16 * j, LANES)
                        rows[i, sl] = rows[i, sl] * w

        ring = (rows0, rows1, rows2, rows3)
        gsem = (sem0, sem1)

        def gather(c, rows, sem):
            return pltpu.async_copy(x_sh.at[srcv.at[c]], rows, sem)

        def wait_gather(c, rows, sem):
            pltpu.make_async_copy(x_sh.at[srcv.at[c]], rows, sem).wait()

        def scatter(rows, c):
            pltpu.async_copy(rows, acc_sh.at[dstv.at[c]], semW, add=True)

        def wait_scatter(rows, c):
            pltpu.make_async_copy(rows, acc_sh.at[dstv.at[c]], semW).wait()

        def stage_body(s, carry):
            sl = pl.ds(s * STAGE_CHUNKS, STAGE_CHUNKS)
            d1 = pltpu.async_copy(src_hbm.at[sid, sl], srcv, semS)
            d2 = pltpu.async_copy(dst_hbm.at[sid, sl], dstv, semS)
            d3 = pltpu.async_copy(rf_hbm.at[sid, sl], rfv, semS)
            d1.wait()
            d2.wait()
            d3.wait()

            gather(0, ring[0], gsem[0])
            gather(1, ring[1], gsem[1])

            def quad_body(q, c2):
                base = 4 * q
                for k in range(4):
                    c = base + k
                    wait_gather(c, ring[k], gsem[k % 2])
                    scale(ring[k], c)
                    scatter(ring[k], c)
                    nk = (k + 2) % 4
                    if k < 2:
                        @pl.when(q > 0)
                        def _():
                            wait_scatter(ring[nk], c - 2)
                        gather(c + 2, ring[nk], gsem[k % 2])
                    else:
                        @pl.when(q < STAGE_CHUNKS // 4 - 1)
                        def _():
                            wait_scatter(ring[nk], c - 2)
                            gather(c + 2, ring[nk], gsem[k % 2])
                return c2

            lax.fori_loop(0, STAGE_CHUNKS // 4, quad_body, 0)
            wait_scatter(ring[0], STAGE_CHUNKS - 4)
            wait_scatter(ring[1], STAGE_CHUNKS - 3)
            wait_scatter(ring[2], STAGE_CHUNKS - 2)
            wait_scatter(ring[3], STAGE_CHUNKS - 1)
            return carry

        lax.fori_loop(0, NSTAGE, stage_body, 0)
        plsc.subcore_barrier()

        pltpu.sync_copy(acc_sh.at[pl.ds(base_row, ROWS_PER_TILE)],
                        out_hbm.at[cid, pl.ds(base_row, ROWS_PER_TILE)])

    return k(src, dst, rf, xcols)


def _gelu(v):
    return 0.5 * v * (1.0 + lax.erf(v * 0.7071067811865476))


def _layer_norm(v, g, b):
    mu = jnp.mean(v, axis=-1, keepdims=True)
    var = jnp.mean(jnp.square(v - mu), axis=-1, keepdims=True)
    return (v - mu) * lax.rsqrt(var + 1e-5) * g + b


def _dense_body(x_ref, ap_ref, wm_ref, w1_ref, b1_ref, w2_ref, b2_ref,
                g1_ref, be1_ref, g2_ref, be2_ref, o_ref):
    cdims = (((1,), (1,)), ((), ()))
    wm = wm_ref[...]
    agg = (lax.dot_general(ap_ref[0], wm[:, :HC], cdims,
                           preferred_element_type=jnp.float32)
           + lax.dot_general(ap_ref[1], wm[:, HC:], cdims,
                             preferred_element_type=jnp.float32))
    t = x_ref[...] + _gelu(agg)
    u = _layer_norm(t, g1_ref[...], be1_ref[...])
    h = _gelu(lax.dot_general(u, w1_ref[...], cdims,
                              preferred_element_type=jnp.float32) + b1_ref[...])
    h2 = lax.dot_general(h, w2_ref[...], cdims,
                         preferred_element_type=jnp.float32) + b2_ref[...]
    o_ref[...] = _layer_norm(u + h2, g2_ref[...], be2_ref[...])


def _tc_dense(x2, halves, W_msg, W1, b1, W2, b2, g1, be1, g2, be2, interpret=False):
    R = 1000
    grid = (N // R,)
    row_spec = pl.BlockSpec((R, D), lambda i: (i, 0))

    def whole(shape):
        return pl.BlockSpec(shape, lambda i: tuple(0 for _ in shape))

    return pl.pallas_call(
        _dense_body,
        grid=grid,
        in_specs=[
            row_spec,
            pl.BlockSpec((2, R, HC), lambda i: (0, i, 0)),
            whole((D, D)), whole((2 * D, D)), whole((1, 2 * D)),
            whole((D, 2 * D)), whole((1, D)),
            whole((1, D)), whole((1, D)), whole((1, D)), whole((1, D)),
        ],
        out_specs=row_spec,
        out_shape=jax.ShapeDtypeStruct((N, D), jnp.float32),
        interpret=interpret,
    )(x2, halves, W_msg, W1, b1.reshape(1, -1), W2,
      b2.reshape(1, -1), g1.reshape(1, -1), be1.reshape(1, -1),
      g2.reshape(1, -1), be2.reshape(1, -1))


def kernel(x, edge_index, routing_factor, W_msg, W1, b1, W2, b2,
           gamma1, beta1, gamma2, beta2):
    x2 = x[0]
    npad_e = EPAD - E
    shp = (NUM_SUBCORES, NCHUNK, CHUNK)
    src3 = jnp.concatenate(
        [edge_index[0], jnp.zeros((npad_e,), jnp.int32)]).reshape(shp)
    dst3 = jnp.concatenate(
        [edge_index[1], jnp.full((npad_e,), NPAD - 8, jnp.int32)]).reshape(shp)
    rf3 = jnp.concatenate(
        [routing_factor, jnp.zeros((npad_e,), jnp.float32)]).reshape(shp)
    xcols = jnp.stack([x2[:, :HC], x2[:, HC:]])

    halves = _sc_scatter_halves(src3, dst3, rf3, xcols)
    out = _tc_dense(x2, halves, W_msg, W1, b1, W2, b2,
                    gamma1, beta1, gamma2, beta2)
    return out[None]


# speedup vs baseline: 1.1343x; 1.1343x over previous
import functools

import jax
import jax.numpy as jnp
from jax import lax
from jax.experimental import pallas as pl
from jax.experimental.pallas import tpu as pltpu
from jax.experimental.pallas import tpu_sc as plsc

N = 10000
D = 128
HC = D // 2
E = 320000
NUM_CORES = 2
NUM_SUBCORES = 16
CHUNK = 64
NCHUNK = 320
EDGES_PER_T = NCHUNK * CHUNK
EPAD = NUM_SUBCORES * EDGES_PER_T
NPAD = 10112
ROWS_PER_TILE = NPAD // NUM_SUBCORES
LANES = 16
STAGE_CHUNKS = 32
NSTAGE = NCHUNK // STAGE_CHUNKS
XROWS_PER_TILE = 624
ZROWS = CHUNK


def _sc_scatter_halves(src, dst, rf, xcols):
    mesh = plsc.VectorSubcoreMesh(
        core_axis_name="c", subcore_axis_name="s",
        num_cores=NUM_CORES, num_subcores=NUM_SUBCORES)

    @functools.partial(
        pl.kernel,
        out_type=jax.ShapeDtypeStruct((NUM_CORES, NPAD, HC), jnp.float32),
        mesh=mesh,
        scratch_types=[
            pltpu.VMEM_SHARED((NPAD, HC), jnp.float32),
            pltpu.VMEM_SHARED((N, HC), jnp.float32),
            pltpu.VMEM((STAGE_CHUNKS, CHUNK), jnp.int32),
            pltpu.VMEM((STAGE_CHUNKS, CHUNK), jnp.int32),
            pltpu.VMEM((STAGE_CHUNKS, CHUNK), jnp.float32),
            pltpu.VMEM((CHUNK, HC), jnp.float32),
            pltpu.VMEM((CHUNK, HC), jnp.float32),
            pltpu.VMEM((CHUNK, HC), jnp.float32),
            pltpu.VMEM((CHUNK, HC), jnp.float32),
            pltpu.SemaphoreType.DMA,
            pltpu.SemaphoreType.DMA,
            pltpu.SemaphoreType.DMA,
            pltpu.SemaphoreType.DMA,
        ],
    )
    def k(src_hbm, dst_hbm, rf_hbm, xc_hbm, out_hbm,
          acc_sh, x_sh, srcv, dstv, rfv, rows0, rows1, rows2, rows3,
          semS, sem0, sem1, semW):
        cid = lax.axis_index("c")
        sid = lax.axis_index("s")

        zeros16 = jnp.zeros((LANES,), jnp.float32)

        def zrow(i, carry):
            for j in range(HC // LANES):
                rows0[i, pl.ds(j * LANES, LANES)] = zeros16
            return carry

        lax.fori_loop(0, ZROWS, zrow, 0)
        base_row = sid * ROWS_PER_TILE
        for t in range(ROWS_PER_TILE // ZROWS):
            pltpu.sync_copy(rows0, acc_sh.at[pl.ds(base_row + t * ZROWS, ZROWS)])
        pltpu.sync_copy(
            rows0.at[pl.ds(0, ROWS_PER_TILE % ZROWS)],
            acc_sh.at[pl.ds(base_row + (ROWS_PER_TILE // ZROWS) * ZROWS,
                            ROWS_PER_TILE % ZROWS)])

        xsl = pl.ds(sid * XROWS_PER_TILE, XROWS_PER_TILE)
        pltpu.sync_copy(xc_hbm.at[cid, xsl], x_sh.at[xsl])

        @pl.when(sid == 0)
        def _():
            tail = pl.ds(NUM_SUBCORES * XROWS_PER_TILE,
                         N - NUM_SUBCORES * XROWS_PER_TILE)
            pltpu.sync_copy(xc_hbm.at[cid, tail], x_sh.at[tail])

        plsc.subcore_barrier()

        def scale(rows, c):
            for g in range(CHUNK // LANES):
                wv = rfv[c, pl.ds(g * LANES, LANES)]
                for l in range(LANES):
                    w = wv[l]
                    i = g * LANES + l
                    for j in range(HC // LANES):
                        sl = pl.ds(16 * j, LANES)
                        rows[i, sl] = rows[i, sl] * w

        ring = (rows0, rows1, rows2, rows3)
        gsem = (sem0, sem1)

        def gather(c, rows, sem):
            return pltpu.async_copy(x_sh.at[srcv.at[c]], rows, sem)

        def wait_gather(c, rows, sem):
            pltpu.make_async_copy(x_sh.at[srcv.at[c]], rows, sem).wait()

        def scatter(rows, c):
            pltpu.async_copy(rows, acc_sh.at[dstv.at[c]], semW, add=True)

        def wait_scatter(rows, c):
            pltpu.make_async_copy(rows, acc_sh.at[dstv.at[c]], semW).wait()

        def stage_body(s, carry):
            sl = pl.ds(s * STAGE_CHUNKS, STAGE_CHUNKS)
            d1 = pltpu.async_copy(src_hbm.at[sid, sl], srcv, semS)
            d2 = pltpu.async_copy(dst_hbm.at[sid, sl], dstv, semS)
            d3 = pltpu.async_copy(rf_hbm.at[sid, sl], rfv, semS)
            d1.wait()
            d2.wait()
            d3.wait()

            gather(0, ring[0], gsem[0])
            gather(1, ring[1], gsem[1])

            def quad_body(q, c2):
                base = 4 * q
                for k in range(4):
                    c = base + k
                    wait_gather(c, ring[k], gsem[k % 2])
                    scale(ring[k], c)
                    scatter(ring[k], c)
                    nk = (k + 2) % 4
                    if k < 2:
                        @pl.when(q > 0)
                        def _():
                            wait_scatter(ring[nk], c - 2)
                        gather(c + 2, ring[nk], gsem[k % 2])
                    else:
                        @pl.when(q < STAGE_CHUNKS // 4 - 1)
                        def _():
                            wait_scatter(ring[nk], c - 2)
                            gather(c + 2, ring[nk], gsem[k % 2])
                return c2

            lax.fori_loop(0, STAGE_CHUNKS // 4, quad_body, 0)
            wait_scatter(ring[0], STAGE_CHUNKS - 4)
            wait_scatter(ring[1], STAGE_CHUNKS - 3)
            wait_scatter(ring[2], STAGE_CHUNKS - 2)
            wait_scatter(ring[3], STAGE_CHUNKS - 1)
            return carry

        lax.fori_loop(0, NSTAGE, stage_body, 0)
        plsc.subcore_barrier()

        pltpu.sync_copy(acc_sh.at[pl.ds(base_row, ROWS_PER_TILE)],
                        out_hbm.at[cid, pl.ds(base_row, ROWS_PER_TILE)])

    return k(src, dst, rf, xcols)


def _gelu(v):
    return 0.5 * v * (1.0 + lax.erf(v * 0.7071067811865476))


def _layer_norm(v, g, b):
    mu = jnp.mean(v, axis=-1, keepdims=True)
    var = jnp.mean(jnp.square(v - mu), axis=-1, keepdims=True)
    return (v - mu) * lax.rsqrt(var + 1e-5) * g + b


def _dense_body(x_ref, ap_ref, wm_ref, w1_ref, b1_ref, w2_ref, b2_ref,
                g1_ref, be1_ref, g2_ref, be2_ref, o_ref):
    cdims = (((1,), (1,)), ((), ()))
    wm = wm_ref[...]
    agg = (lax.dot_general(ap_ref[0], wm[:, :HC], cdims,
                           preferred_element_type=jnp.float32)
           + lax.dot_general(ap_ref[1], wm[:, HC:], cdims,
                             preferred_element_type=jnp.float32))
    t = x_ref[...] + _gelu(agg)
    u = _layer_norm(t, g1_ref[...], be1_ref[...])
    h = _gelu(lax.dot_general(u, w1_ref[...], cdims,
                              preferred_element_type=jnp.float32) + b1_ref[...])
    h2 = lax.dot_general(h, w2_ref[...], cdims,
                         preferred_element_type=jnp.float32) + b2_ref[...]
    o_ref[...] = _layer_norm(u + h2, g2_ref[...], be2_ref[...])


def _tc_dense(x2, halves, W_msg, W1, b1, W2, b2, g1, be1, g2, be2, interpret=False):
    R = 1000
    grid = (N // R,)
    row_spec = pl.BlockSpec((R, D), lambda i: (i, 0))

    def whole(shape):
        return pl.BlockSpec(shape, lambda i: tuple(0 for _ in shape))

    return pl.pallas_call(
        _dense_body,
        grid=grid,
        in_specs=[
            row_spec,
            pl.BlockSpec((2, R, HC), lambda i: (0, i, 0)),
            whole((D, D)), whole((2 * D, D)), whole((1, 2 * D)),
            whole((D, 2 * D)), whole((1, D)),
            whole((1, D)), whole((1, D)), whole((1, D)), whole((1, D)),
        ],
        out_specs=row_spec,
        out_shape=jax.ShapeDtypeStruct((N, D), jnp.float32),
        interpret=interpret,
    )(x2, halves, W_msg, W1, b1.reshape(1, -1), W2,
      b2.reshape(1, -1), g1.reshape(1, -1), be1.reshape(1, -1),
      g2.reshape(1, -1), be2.reshape(1, -1))


def kernel(x, edge_index, routing_factor, W_msg, W1, b1, W2, b2,
           gamma1, beta1, gamma2, beta2):
    x2 = x[0]
    npad_e = EPAD - E
    shp = (NUM_SUBCORES, NCHUNK, CHUNK)
    src3 = jnp.concatenate(
        [edge_index[0], jnp.zeros((npad_e,), jnp.int32)]).reshape(shp)
    dst3 = jnp.concatenate(
        [edge_index[1], jnp.full((npad_e,), NPAD - 8, jnp.int32)]).reshape(shp)
    rf3 = jnp.concatenate(
        [routing_factor, jnp.zeros((npad_e,), jnp.float32)]).reshape(shp)
    xcols = jnp.stack([x2[:, :HC], x2[:, HC:]])

    halves = _sc_scatter_halves(src3, dst3, rf3, xcols)
    out = _tc_dense(x2, halves, W_msg, W1, b1, W2, b2,
                    gamma1, beta1, gamma2, beta2)
    return out[None]


# speedup vs baseline: 1.1624x; 1.0248x over previous
import functools

import jax
import jax.numpy as jnp
from jax import lax
from jax.experimental import pallas as pl
from jax.experimental.pallas import tpu as pltpu
from jax.experimental.pallas import tpu_sc as plsc

N = 10000
D = 128
HC = D // 2
E = 320000
NUM_CORES = 2
NUM_SUBCORES = 16
CHUNK = 64
NCHUNK = 320
EDGES_PER_T = NCHUNK * CHUNK
EPAD = NUM_SUBCORES * EDGES_PER_T
NPAD = 10112
ROWS_PER_TILE = NPAD // NUM_SUBCORES
LANES = 16
STAGE_CHUNKS = 32
NSTAGE = NCHUNK // STAGE_CHUNKS
XROWS_PER_TILE = 624
ZROWS = CHUNK


def _sc_scatter_halves(src, dst, rf, xcols):
    mesh = plsc.VectorSubcoreMesh(
        core_axis_name="c", subcore_axis_name="s",
        num_cores=NUM_CORES, num_subcores=NUM_SUBCORES)

    @functools.partial(
        pl.kernel,
        out_type=jax.ShapeDtypeStruct((NUM_CORES, NPAD, HC), jnp.float32),
        mesh=mesh,
        scratch_types=[
            pltpu.VMEM_SHARED((NPAD, HC), jnp.float32),
            pltpu.VMEM_SHARED((N, HC), jnp.float32),
            pltpu.VMEM((STAGE_CHUNKS, CHUNK), jnp.int32),
            pltpu.VMEM((STAGE_CHUNKS, CHUNK), jnp.int32),
            pltpu.VMEM((STAGE_CHUNKS, CHUNK), jnp.float32),
            pltpu.VMEM((CHUNK, HC), jnp.float32),
            pltpu.VMEM((CHUNK, HC), jnp.float32),
            pltpu.VMEM((CHUNK, HC), jnp.float32),
            pltpu.VMEM((CHUNK, HC), jnp.float32),
            pltpu.SemaphoreType.DMA,
            pltpu.SemaphoreType.DMA,
            pltpu.SemaphoreType.DMA,
            pltpu.SemaphoreType.DMA,
        ],
    )
    def k(src_hbm, dst_hbm, rf_hbm, xc_hbm, out_hbm,
          acc_sh, x_sh, srcv, dstv, rfv, rows0, rows1, rows2, rows3,
          semS, sem0, sem1, semW):
        cid = lax.axis_index("c")
        sid = lax.axis_index("s")

        zeros16 = jnp.zeros((LANES,), jnp.float32)

        def zrow(i, carry):
            for j in range(HC // LANES):
                rows0[i, pl.ds(j * LANES, LANES)] = zeros16
            return carry

        lax.fori_loop(0, ZROWS, zrow, 0)
        base_row = sid * ROWS_PER_TILE
        for t in range(ROWS_PER_TILE // ZROWS):
            pltpu.sync_copy(rows0, acc_sh.at[pl.ds(base_row + t * ZROWS, ZROWS)])
        pltpu.sync_copy(
            rows0.at[pl.ds(0, ROWS_PER_TILE % ZROWS)],
            acc_sh.at[pl.ds(base_row + (ROWS_PER_TILE // ZROWS) * ZROWS,
                            ROWS_PER_TILE % ZROWS)])

        xsl = pl.ds(sid * XROWS_PER_TILE, XROWS_PER_TILE)
        pltpu.sync_copy(xc_hbm.at[cid, xsl], x_sh.at[xsl])

        @pl.when(sid == 0)
        def _():
            tail = pl.ds(NUM_SUBCORES * XROWS_PER_TILE,
                         N - NUM_SUBCORES * XROWS_PER_TILE)
            pltpu.sync_copy(xc_hbm.at[cid, tail], x_sh.at[tail])

        plsc.subcore_barrier()

        def scale(rows, c):
            for g in range(CHUNK // LANES):
                wv = rfv[c, pl.ds(g * LANES, LANES)]
                for l in range(LANES):
                    w = wv[l]
                    i = g * LANES + l
                    for j in range(HC // LANES):
                        sl = pl.ds(16 * j, LANES)
                        rows[i, sl] = rows[i, sl] * w

        ring = (rows0, rows1, rows2, rows3)
        gsem = (sem0, sem1)

        def gather(c, rows, sem):
            return pltpu.async_copy(x_sh.at[srcv.at[c]], rows, sem)

        def wait_gather(c, rows, sem):
            pltpu.make_async_copy(x_sh.at[srcv.at[c]], rows, sem).wait()

        def scatter(rows, c):
            pltpu.async_copy(rows, acc_sh.at[dstv.at[c]], semW, add=True)

        def wait_scatter(rows, c):
            pltpu.make_async_copy(rows, acc_sh.at[dstv.at[c]], semW).wait()

        def stage_body(s, carry):
            sl = pl.ds(s * STAGE_CHUNKS, STAGE_CHUNKS)
            d1 = pltpu.async_copy(src_hbm.at[sid, sl], srcv, semS)
            d2 = pltpu.async_copy(dst_hbm.at[sid, sl], dstv, semS)
            d3 = pltpu.async_copy(rf_hbm.at[sid, sl], rfv, semS)
            d1.wait()
            d2.wait()
            d3.wait()

            gather(0, ring[0], gsem[0])
            gather(1, ring[1], gsem[1])

            def quad_body(q, c2):
                base = 4 * q
                for k in range(4):
                    c = base + k
                    wait_gather(c, ring[k], gsem[k % 2])
                    nk = (k + 2) % 4
                    if k < 2:
                        @pl.when(q > 0)
                        def _():
                            wait_scatter(ring[nk], c - 2)
                        gather(c + 2, ring[nk], gsem[k % 2])
                    else:
                        @pl.when(q < STAGE_CHUNKS // 4 - 1)
                        def _():
                            wait_scatter(ring[nk], c - 2)
                            gather(c + 2, ring[nk], gsem[k % 2])
                    scale(ring[k], c)
                    scatter(ring[k], c)
                return c2

            lax.fori_loop(0, STAGE_CHUNKS // 4, quad_body, 0)
            wait_scatter(ring[0], STAGE_CHUNKS - 4)
            wait_scatter(ring[1], STAGE_CHUNKS - 3)
            wait_scatter(ring[2], STAGE_CHUNKS - 2)
            wait_scatter(ring[3], STAGE_CHUNKS - 1)
            return carry

        lax.fori_loop(0, NSTAGE, stage_body, 0)
        plsc.subcore_barrier()

        pltpu.sync_copy(acc_sh.at[pl.ds(base_row, ROWS_PER_TILE)],
                        out_hbm.at[cid, pl.ds(base_row, ROWS_PER_TILE)])

    return k(src, dst, rf, xcols)


def _gelu(v):
    return 0.5 * v * (1.0 + lax.erf(v * 0.7071067811865476))


def _layer_norm(v, g, b):
    mu = jnp.mean(v, axis=-1, keepdims=True)
    var = jnp.mean(jnp.square(v - mu), axis=-1, keepdims=True)
    return (v - mu) * lax.rsqrt(var + 1e-5) * g + b


def _dense_body(x_ref, ap_ref, wm_ref, w1_ref, b1_ref, w2_ref, b2_ref,
                g1_ref, be1_ref, g2_ref, be2_ref, o_ref):
    cdims = (((1,), (1,)), ((), ()))
    wm = wm_ref[...]
    agg = (lax.dot_general(ap_ref[0], wm[:, :HC], cdims,
                           preferred_element_type=jnp.float32)
           + lax.dot_general(ap_ref[1], wm[:, HC:], cdims,
                             preferred_element_type=jnp.float32))
    t = x_ref[...] + _gelu(agg)
    u = _layer_norm(t, g1_ref[...], be1_ref[...])
    h = _gelu(lax.dot_general(u, w1_ref[...], cdims,
                              preferred_element_type=jnp.float32) + b1_ref[...])
    h2 = lax.dot_general(h, w2_ref[...], cdims,
                         preferred_element_type=jnp.float32) + b2_ref[...]
    o_ref[...] = _layer_norm(u + h2, g2_ref[...], be2_ref[...])


def _tc_dense(x2, halves, W_msg, W1, b1, W2, b2, g1, be1, g2, be2, interpret=False):
    R = 1000
    grid = (N // R,)
    row_spec = pl.BlockSpec((R, D), lambda i: (i, 0))

    def whole(shape):
        return pl.BlockSpec(shape, lambda i: tuple(0 for _ in shape))

    return pl.pallas_call(
        _dense_body,
        grid=grid,
        in_specs=[
            row_spec,
            pl.BlockSpec((2, R, HC), lambda i: (0, i, 0)),
            whole((D, D)), whole((2 * D, D)), whole((1, 2 * D)),
            whole((D, 2 * D)), whole((1, D)),
            whole((1, D)), whole((1, D)), whole((1, D)), whole((1, D)),
        ],
        out_specs=row_spec,
        out_shape=jax.ShapeDtypeStruct((N, D), jnp.float32),
        interpret=interpret,
    )(x2, halves, W_msg, W1, b1.reshape(1, -1), W2,
      b2.reshape(1, -1), g1.reshape(1, -1), be1.reshape(1, -1),
      g2.reshape(1, -1), be2.reshape(1, -1))


def kernel(x, edge_index, routing_factor, W_msg, W1, b1, W2, b2,
           gamma1, beta1, gamma2, beta2):
    x2 = x[0]
    npad_e = EPAD - E
    shp = (NUM_SUBCORES, NCHUNK, CHUNK)
    src3 = jnp.concatenate(
        [edge_index[0], jnp.zeros((npad_e,), jnp.int32)]).reshape(shp)
    dst3 = jnp.concatenate(
        [edge_index[1], jnp.full((npad_e,), NPAD - 8, jnp.int32)]).reshape(shp)
    rf3 = jnp.concatenate(
        [routing_factor, jnp.zeros((npad_e,), jnp.float32)]).reshape(shp)
    xcols = jnp.stack([x2[:, :HC], x2[:, HC:]])

    halves = _sc_scatter_halves(src3, dst3, rf3, xcols)
    out = _tc_dense(x2, halves, W_msg, W1, b1, W2, b2,
                    gamma1, beta1, gamma2, beta2)
    return out[None]
